# baseline (speedup 1.0000x reference)
import functools
import math

import jax
import jax.numpy as jnp
from jax import lax
from jax.experimental import pallas as pl
from jax.experimental.pallas import tpu as pltpu
from jax.experimental.pallas import tpu_sc as plsc

_NUM_CORES = 2
_NUM_SUBCORES = 16
_NUM_WORKERS = _NUM_CORES * _NUM_SUBCORES
_LANES = 16


def _pe_table(seq_len, d_model):
    qblk = 64
    rsz = 64
    blk = qblk * rsz
    neg_log_base = -math.log(10000.0) / d_model
    half_pi = math.pi / 2.0

    def body(o_ref, sinb_ref, cosb_ref):
        i = pl.program_id(0)
        col = lax.broadcasted_iota(jnp.int32, (1, d_model), 1)
        w = jnp.exp(col.astype(jnp.float32) * neg_log_base)
        shift = (col % 2).astype(jnp.float32) * half_pi

        @pl.when(i == 0)
        def _():
            r = lax.broadcasted_iota(jnp.int32, (rsz, d_model), 0)
            arg = r.astype(jnp.float32) * w
            sinb_ref[...] = jnp.sin(arg)
            cosb_ref[...] = jnp.cos(arg)

        q = lax.broadcasted_iota(jnp.int32, (qblk, 1, d_model), 0) + i * qblk
        a = q.astype(jnp.float32) * (float(rsz) * w[None]) + shift[None]
        sin_a = jnp.sin(a)
        cos_a = jnp.cos(a)
        val = sin_a * cosb_ref[...][None] + cos_a * sinb_ref[...][None]
        o_ref[...] = val.reshape(blk, d_model)

    return pl.pallas_call(
        body,
        out_shape=jax.ShapeDtypeStruct((seq_len, d_model), jnp.float32),
        grid=(seq_len // blk,),
        out_specs=pl.BlockSpec((blk, d_model), lambda i: (i, 0)),
        scratch_shapes=[
            pltpu.VMEM((rsz, d_model), jnp.float32),
            pltpu.VMEM((rsz, d_model), jnp.float32),
        ],
    )()


def _sc_gather_add(table, x, pe, batch, seq_len, d_model):
    pos_per_worker = seq_len // _NUM_WORKERS
    chunk = 16
    steps = batch * (pos_per_worker // chunk)
    chunks_per_batch = pos_per_worker // chunk
    groups = d_model // _LANES
    n_rows = batch * seq_len
    mesh = plsc.VectorSubcoreMesh(core_axis_name="c", subcore_axis_name="s")

    n_bufs = 2 * batch
    scratch_types = (
        [pltpu.VMEM((batch, pos_per_worker), jnp.int32)]
        + [pltpu.VMEM((chunk, d_model), jnp.float32)] * 2
        + [pltpu.VMEM((chunk, d_model), jnp.float32)] * n_bufs
        + [pltpu.SemaphoreType.DMA] * (3 + 2 * n_bufs)
    )

    @functools.partial(
        pl.kernel,
        mesh=mesh,
        out_type=jax.ShapeDtypeStruct((n_rows, d_model), table.dtype),
        scratch_types=scratch_types,
    )
    def gather_kernel(table_hbm, idx_hbm, pe_hbm, out_hbm, idx_v, *sc):
        pe_bufs = sc[0:2]
        rows = sc[2:2 + n_bufs]
        idx_sem = sc[2 + n_bufs]
        pe_sems = sc[3 + n_bufs:5 + n_bufs]
        g_sems = sc[5 + n_bufs:5 + 2 * n_bufs]
        s_sems = sc[5 + 2 * n_bufs:5 + 3 * n_bufs]
        wid = lax.axis_index("s") * _NUM_CORES + lax.axis_index("c")
        pbase = wid * pos_per_worker

        def fire_pe(ci, par):
            return pltpu.async_copy(
                pe_hbm.at[pl.ds(pbase + ci * chunk, chunk)], pe_bufs[par],
                pe_sems[par])

        def fire_gather(ci, k):
            return pltpu.async_copy(
                table_hbm.at[idx_v.at[k % batch, pl.ds(ci * chunk, chunk)]],
                rows[k], g_sems[k])

        def out_slice(ci, b):
            return out_hbm.at[
                pl.ds(b * seq_len + pbase + ci * chunk, chunk)]

        fire_pe(0, 0)
        idx_cds = [
            pltpu.async_copy(
                idx_hbm.at[b, pl.ds(pbase, pos_per_worker)],
                idx_v.at[b], idx_sem)
            for b in range(batch)
        ]
        for cd in idx_cds:
            cd.wait()
        for b in range(batch):
            fire_gather(0, b)

        @pl.loop(0, chunks_per_batch, step=2)
        def _(ci2):
          for par in range(2):
            ci = ci2 + par
            cur = rows[batch * par:batch * par + batch]
            nxt = rows[batch * (1 - par):batch * (1 - par) + batch]
            cur_g = g_sems[batch * par:batch * par + batch]
            nxt_g = g_sems[batch * (1 - par):batch * (1 - par) + batch]
            cur_s = s_sems[batch * par:batch * par + batch]
            nxt_s = s_sems[batch * (1 - par):batch * (1 - par) + batch]

            @pl.when(ci + 1 < chunks_per_batch)
            def _():
                fire_pe(ci + 1, 1 - par)
                for b in range(batch):
                    @pl.when(ci >= 1)
                    def _():
                        pltpu.make_async_copy(
                            nxt[b], out_slice(ci, b),
                            nxt_s[b]).wait()
                    fire_gather(ci + 1, batch * (1 - par) + b)

            pltpu.make_async_copy(
                pe_hbm.at[pl.ds(pbase, chunk)], pe_bufs[par],
                pe_sems[par]).wait()
            pe_v = pe_bufs[par]

            for b in range(batch):
                pltpu.make_async_copy(
                    table_hbm.at[idx_v.at[0, pl.ds(0, chunk)]], cur[b],
                    cur_g[b]).wait()

                @pl.loop(0, chunk, step=4)
                def _(rr):
                    for r0 in range(4):
                        for g in range(groups):
                            sl = pl.ds(g * _LANES, _LANES)
                            plsc.addupdate(cur[b].at[rr + r0, sl],
                                           pe_v[rr + r0, sl])

                pltpu.async_copy(cur[b], out_slice(ci, b), cur_s[b])

        for ci in (chunks_per_batch - 2, chunks_per_batch - 1):
            par = ci % 2
            for b in range(batch):
                pltpu.make_async_copy(
                    rows[batch * par + b], out_slice(ci, b),
                    s_sems[batch * par + b]).wait()

    return gather_kernel(table, x, pe)


def kernel(x, table):
    batch, seq_len = x.shape
    d_model = table.shape[1]
    pe = _pe_table(seq_len, d_model)
    out = _sc_gather_add(table, x, pe, batch, seq_len, d_model)
    return out.reshape(batch, seq_len, d_model)


# speedup vs baseline: 1.0297x; 1.0297x over previous
import functools
import math

import jax
import jax.numpy as jnp
from jax import lax
from jax.experimental import pallas as pl
from jax.experimental.pallas import tpu as pltpu
from jax.experimental.pallas import tpu_sc as plsc

_NUM_CORES = 2
_NUM_SUBCORES = 16
_NUM_WORKERS = _NUM_CORES * _NUM_SUBCORES
_LANES = 16


def _pe_table(seq_len, d_model):
    qblk = 16
    rsz = 64
    blk = qblk * rsz
    neg_log_base = -math.log(10000.0) / d_model
    half_pi = math.pi / 2.0

    def body(o_ref, sinb_ref, cosb_ref):
        i = pl.program_id(0)
        col = lax.broadcasted_iota(jnp.int32, (1, d_model), 1)
        w = jnp.exp(col.astype(jnp.float32) * neg_log_base)
        shift = (col % 2).astype(jnp.float32) * half_pi

        @pl.when(i == 0)
        def _():
            r = lax.broadcasted_iota(jnp.int32, (rsz, d_model), 0)
            arg = r.astype(jnp.float32) * w
            sinb_ref[...] = jnp.sin(arg)
            cosb_ref[...] = jnp.cos(arg)

        q = lax.broadcasted_iota(jnp.int32, (qblk, 1, d_model), 0) + i * qblk
        a = q.astype(jnp.float32) * (float(rsz) * w[None]) + shift[None]
        sin_a = jnp.sin(a)
        cos_a = jnp.cos(a)
        val = sin_a * cosb_ref[...][None] + cos_a * sinb_ref[...][None]
        o_ref[...] = val.reshape(blk, d_model)

    return pl.pallas_call(
        body,
        out_shape=jax.ShapeDtypeStruct((seq_len, d_model), jnp.float32),
        grid=(seq_len // blk,),
        out_specs=pl.BlockSpec((blk, d_model), lambda i: (i, 0)),
        scratch_shapes=[
            pltpu.VMEM((rsz, d_model), jnp.float32),
            pltpu.VMEM((rsz, d_model), jnp.float32),
        ],
    )()


def _sc_gather_add(table, x, pe, batch, seq_len, d_model):
    pos_per_worker = seq_len // _NUM_WORKERS
    chunk = 16
    steps = batch * (pos_per_worker // chunk)
    chunks_per_batch = pos_per_worker // chunk
    groups = d_model // _LANES
    n_rows = batch * seq_len
    mesh = plsc.VectorSubcoreMesh(core_axis_name="c", subcore_axis_name="s")

    n_bufs = 2 * batch
    scratch_types = (
        [pltpu.VMEM((batch, pos_per_worker), jnp.int32)]
        + [pltpu.VMEM((chunk, d_model), jnp.float32)] * 2
        + [pltpu.VMEM((chunk, d_model), jnp.float32)] * n_bufs
        + [pltpu.SemaphoreType.DMA] * (3 + 2 * n_bufs)
    )

    @functools.partial(
        pl.kernel,
        mesh=mesh,
        out_type=jax.ShapeDtypeStruct((n_rows, d_model), table.dtype),
        scratch_types=scratch_types,
    )
    def gather_kernel(table_hbm, idx_hbm, pe_hbm, out_hbm, idx_v, *sc):
        pe_bufs = sc[0:2]
        rows = sc[2:2 + n_bufs]
        idx_sem = sc[2 + n_bufs]
        pe_sems = sc[3 + n_bufs:5 + n_bufs]
        g_sems = sc[5 + n_bufs:5 + 2 * n_bufs]
        s_sems = sc[5 + 2 * n_bufs:5 + 3 * n_bufs]
        wid = lax.axis_index("s") * _NUM_CORES + lax.axis_index("c")
        pbase = wid * pos_per_worker

        def fire_pe(ci, par):
            return pltpu.async_copy(
                pe_hbm.at[pl.ds(pbase + ci * chunk, chunk)], pe_bufs[par],
                pe_sems[par])

        def fire_gather(ci, k):
            return pltpu.async_copy(
                table_hbm.at[idx_v.at[k % batch, pl.ds(ci * chunk, chunk)]],
                rows[k], g_sems[k])

        def out_slice(ci, b):
            return out_hbm.at[
                pl.ds(b * seq_len + pbase + ci * chunk, chunk)]

        fire_pe(0, 0)
        idx_cds = [
            pltpu.async_copy(
                idx_hbm.at[b, pl.ds(pbase, pos_per_worker)],
                idx_v.at[b], idx_sem)
            for b in range(batch)
        ]
        for cd in idx_cds:
            cd.wait()
        for b in range(batch):
            fire_gather(0, b)

        @pl.loop(0, chunks_per_batch, step=2)
        def _(ci2):
          for par in range(2):
            ci = ci2 + par
            cur = rows[batch * par:batch * par + batch]
            nxt = rows[batch * (1 - par):batch * (1 - par) + batch]
            cur_g = g_sems[batch * par:batch * par + batch]
            nxt_g = g_sems[batch * (1 - par):batch * (1 - par) + batch]
            cur_s = s_sems[batch * par:batch * par + batch]
            nxt_s = s_sems[batch * (1 - par):batch * (1 - par) + batch]

            @pl.when(ci + 1 < chunks_per_batch)
            def _():
                fire_pe(ci + 1, 1 - par)
                for b in range(batch):
                    @pl.when(ci >= 1)
                    def _():
                        pltpu.make_async_copy(
                            nxt[b], out_slice(ci, b),
                            nxt_s[b]).wait()
                    fire_gather(ci + 1, batch * (1 - par) + b)

            pltpu.make_async_copy(
                pe_hbm.at[pl.ds(pbase, chunk)], pe_bufs[par],
                pe_sems[par]).wait()
            pe_v = pe_bufs[par]

            for b in range(batch):
                pltpu.make_async_copy(
                    table_hbm.at[idx_v.at[0, pl.ds(0, chunk)]], cur[b],
                    cur_g[b]).wait()

                @pl.loop(0, chunk, step=4)
                def _(rr):
                    for r0 in range(4):
                        for g in range(groups):
                            sl = pl.ds(g * _LANES, _LANES)
                            plsc.addupdate(cur[b].at[rr + r0, sl],
                                           pe_v[rr + r0, sl])

                pltpu.async_copy(cur[b], out_slice(ci, b), cur_s[b])

        for ci in (chunks_per_batch - 2, chunks_per_batch - 1):
            par = ci % 2
            for b in range(batch):
                pltpu.make_async_copy(
                    rows[batch * par + b], out_slice(ci, b),
                    s_sems[batch * par + b]).wait()

    return gather_kernel(table, x, pe)


def kernel(x, table):
    batch, seq_len = x.shape
    d_model = table.shape[1]
    pe = _pe_table(seq_len, d_model)
    out = _sc_gather_add(table, x, pe, batch, seq_len, d_model)
    return out.reshape(batch, seq_len, d_model)
